# baseline (speedup 1.0000x reference)
import functools

import jax
import jax.numpy as jnp
from jax import lax
from jax.experimental import pallas as pl
from jax.experimental.pallas import tpu as pltpu
from jax.experimental.pallas import tpu_sc as plsc

N = 10000
NP = 10240
E = 320000
F = 128
H = 128
C = 64
S = 4
SSW = 0.05

NC = 2
NS = 16
NW = NC * NS
EW = E // NW
K = 80
NCH = EW // K
NB = 4
NI = 6
RPT = NP // NS

_HIGH = lax.Precision.DEFAULT


def _elu(v):
    return jnp.where(v > 0, v, jnp.exp(v) - 1.0)


@functools.lru_cache(maxsize=None)
def _make_sc_pass(with_deg):
    mesh = plsc.VectorSubcoreMesh(core_axis_name="c", subcore_axis_name="s",
                                  num_cores=NC, num_subcores=NS)
    out_type = [jax.ShapeDtypeStruct((NC, NP, H), jnp.float32)]
    if with_deg:
        out_type.append(jax.ShapeDtypeStruct((NC, NP), jnp.float32))

    scratch = [
        pltpu.VMEM((NI, 1, K), jnp.int32),
        pltpu.VMEM((NI, 1, K), jnp.int32),
        pltpu.VMEM((NB, K, H), jnp.float32),
        pltpu.VMEM((K,), jnp.float32),
        pltpu.VMEM_SHARED((NP, H), jnp.float32),
        pltpu.VMEM_SHARED((NP,), jnp.float32),
        pltpu.SemaphoreType.DMA,
        pltpu.SemaphoreType.DMA,
        pltpu.SemaphoreType.DMA,
        pltpu.SemaphoreType.DMA,
    ]

    def body(x_hbm, ei_hbm, *rest):
        if with_deg:
            part_hbm, deg_hbm = rest[0], rest[1]
            rest = rest[2:]
        else:
            part_hbm = rest[0]
            deg_hbm = None
            rest = rest[1:]
        sidx, didx, rows, ones, agg_sh, deg_sh, sem_i, sem_g, sem_s, sem_d = rest

        c = lax.axis_index("c")
        s = lax.axis_index("s")
        wid = c * NS + s
        r0 = s * RPT

        ebase = wid * EW

        def stage_idx(j, jx):
            e0 = ebase + j * K
            pltpu.async_copy(ei_hbm.at[pl.ds(e0, K)], sidx.at[jx, 0], sem_i)
            pltpu.async_copy(ei_hbm.at[pl.ds(E + e0, K)], didx.at[jx, 0], sem_i)

        def wait_idx(jx):
            pltpu.make_async_copy(ei_hbm.at[pl.ds(0, K)], sidx.at[jx, 0],
                                  sem_i).wait()
            pltpu.make_async_copy(ei_hbm.at[pl.ds(0, K)], didx.at[jx, 0],
                                  sem_i).wait()

        def issue_gather(jx, q):
            pltpu.async_copy(x_hbm.at[sidx.at[jx, 0]], rows.at[q], sem_g)

        def wait_gather_issue_scatter(jx, q):
            pltpu.make_async_copy(x_hbm.at[sidx.at[jx, 0]], rows.at[q],
                                  sem_g).wait()
            pltpu.async_copy(rows.at[q], agg_sh.at[didx.at[jx, 0]],
                             sem_s, add=True)
            if with_deg:
                pltpu.async_copy(ones, deg_sh.at[didx.at[jx, 0]],
                                 sem_d, add=True)

        def drain_scatter(jx, q):
            pltpu.make_async_copy(rows.at[q], agg_sh.at[didx.at[jx, 0]],
                                  sem_s).wait()
            if with_deg:
                pltpu.make_async_copy(ones, deg_sh.at[didx.at[jx, 0]],
                                      sem_d).wait()

        def chunk_step(js, jr=None, drain=True, stage=True, advance=True):
            if jr is None:
                jr = js
            q = js % NB
            jx = js % NI
            if drain:
                drain_scatter((js - 2) % NI, (js - 2) % NB)
            if stage:
                stage_idx(jr + 3, (js + 3) % NI)
            if advance:
                wait_idx((js + 2) % NI)
                issue_gather((js + 2) % NI, (js + 2) % NB)
            wait_gather_issue_scatter(jx, q)

        stage_idx(0, 0)
        stage_idx(1, 1)
        stage_idx(2, 2)

        @pl.loop(0, K)
        def _(i):
            @pl.loop(0, H, step=16)
            def _(j):
                rows.at[2, i, pl.ds(j, 16)][...] = jnp.zeros((16,), jnp.float32)

        @pl.loop(0, K, step=16)
        def _(j):
            ones.at[pl.ds(j, 16)][...] = jnp.ones((16,), jnp.float32)

        @pl.loop(0, RPT, step=K)
        def _(r):
            pltpu.sync_copy(rows.at[2], agg_sh.at[pl.ds(r0 + r, K)])

        if with_deg:
            @pl.loop(0, RPT, step=128)
            def _(r):
                pltpu.sync_copy(rows.at[2, 0], deg_sh.at[pl.ds(r0 + r, 128)])

        wait_idx(0)
        issue_gather(0, 0)
        wait_idx(1)
        issue_gather(1, 1)
        plsc.subcore_barrier()

        chunk_step(0, drain=False)
        chunk_step(1, drain=False)

        @pl.loop(2, NCH - 3, step=12)
        def _(jb):
            for u in range(12):
                chunk_step(2 + u, jb + u)

        chunk_step(NCH - 3, stage=False)
        chunk_step(NCH - 2, stage=False, advance=False)
        chunk_step(NCH - 1, stage=False, advance=False)
        drain_scatter((NCH - 2) % NI, (NCH - 2) % NB)
        drain_scatter((NCH - 1) % NI, (NCH - 1) % NB)

        plsc.subcore_barrier()

        pltpu.sync_copy(agg_sh.at[pl.ds(r0, RPT)], part_hbm.at[c, pl.ds(r0, RPT)])
        if with_deg:
            pltpu.sync_copy(deg_sh.at[pl.ds(r0, RPT)], deg_hbm.at[c, pl.ds(r0, RPT)])

    return pl.kernel(body, out_type=out_type, mesh=mesh, scratch_types=scratch)


_BT = 5000

_W2 = [pl.BlockSpec((H, H), lambda i: (0, 0))]
_P_SPEC = pl.BlockSpec((NC, _BT, H), lambda i: (0, i, 0))
_D_SPEC = pl.BlockSpec((NC, _BT, 1), lambda i: (0, i, 0))
_ROW_SPEC = pl.BlockSpec((_BT, H), lambda i: (i, 0))
_L_SPEC = pl.BlockSpec((1, 1), lambda i: (0, 0))


def _dot(a, b):
    return jnp.dot(a, b, precision=_HIGH, preferred_element_type=jnp.float32)


def _tc_fuse_body(win_ref, bin_ref, wm_ref, ws_ref, b_ref,
                  wim_ref, wis_ref, bim_ref, bc_ref):
    hi = lax.Precision.HIGHEST
    win = win_ref[...]
    bin_ = bin_ref[...]
    wim_ref[...] = jnp.dot(win, wm_ref[...], precision=hi,
                           preferred_element_type=jnp.float32)
    wis_ref[...] = jnp.dot(win, ws_ref[...], precision=hi,
                           preferred_element_type=jnp.float32)
    bim_ref[...] = jnp.dot(bin_, wm_ref[...], precision=hi,
                           preferred_element_type=jnp.float32)
    bc_ref[...] = (
        jnp.dot(bin_, ws_ref[...], precision=hi,
                preferred_element_type=jnp.float32)
        + b_ref[...]
    )


def _tc_fuse(W_in, b_in2, Wm, Ws, b2):
    w_spec = pl.BlockSpec((H, H), lambda: (0, 0))
    b_spec = pl.BlockSpec((1, H), lambda: (0, 0))
    return pl.pallas_call(
        _tc_fuse_body,
        in_specs=[w_spec, b_spec, w_spec, w_spec, b_spec],
        out_specs=[w_spec, w_spec, b_spec, b_spec],
        out_shape=[
            jax.ShapeDtypeStruct((H, H), jnp.float32),
            jax.ShapeDtypeStruct((H, H), jnp.float32),
            jax.ShapeDtypeStruct((1, H), jnp.float32),
            jax.ShapeDtypeStruct((1, H), jnp.float32),
        ],
    )(W_in, b_in2, Wm, Ws, b2)


def _tc_layer0_body(p_ref, d_ref, h_ref, wim_ref, wis_ref, bim_ref, bc_ref,
                    wss_ref, bss_ref, xn_ref, loss_ref):
    i = pl.program_id(0)
    dsum = d_ref[0] + d_ref[1]
    inv = 1.0 / jnp.maximum(dsum, 1.0)
    mask = jnp.minimum(dsum, 1.0)
    hbar = (p_ref[0] + p_ref[1]) * inv
    xn = _elu(_dot(hbar, wim_ref[...]) + _dot(h_ref[...], wis_ref[...])
              + bim_ref[...] * mask + bc_ref[...])
    xn_ref[...] = xn
    dd = _dot(xn, wss_ref[...]) + bss_ref[...] - h_ref[...]

    @pl.when(i == 0)
    def _():
        loss_ref[...] = jnp.zeros((1, 1), jnp.float32)

    loss_ref[...] = loss_ref[...] + (jnp.sum(dd * dd) * (SSW / (N * F)))[None, None]


def _tc_layer0(p, deg3, h, Wim, Wis, bim, bc, Wss, bss2):
    return pl.pallas_call(
        _tc_layer0_body,
        grid=(N // _BT,),
        in_specs=[
            _P_SPEC,
            _D_SPEC,
            _ROW_SPEC,
            pl.BlockSpec((H, H), lambda i: (0, 0)),
            pl.BlockSpec((H, H), lambda i: (0, 0)),
            pl.BlockSpec((1, H), lambda i: (0, 0)),
            pl.BlockSpec((1, H), lambda i: (0, 0)),
            pl.BlockSpec((H, F), lambda i: (0, 0)),
            pl.BlockSpec((1, F), lambda i: (0, 0)),
        ],
        out_specs=[_ROW_SPEC, _L_SPEC],
        out_shape=[
            jax.ShapeDtypeStruct((N, H), jnp.float32),
            jax.ShapeDtypeStruct((1, 1), jnp.float32),
        ],
    )(p, deg3, h, Wim, Wis, bim, bc, Wss, bss2)


def _tc_layer1_out_body(p_ref, d_ref, x_ref, h_ref, wm_ref, ws_ref, b_ref,
                        wss_ref, bss_ref, wo_ref, bo_ref, lprev_ref,
                        o_ref, loss_ref):
    i = pl.program_id(0)
    sidx = pl.program_id(1)
    dsum = d_ref[0] + d_ref[1]
    inv = 1.0 / jnp.maximum(dsum, 1.0)
    agg = (p_ref[0] + p_ref[1]) * inv
    xn = _elu(_dot(agg, wm_ref[...]) + _dot(x_ref[...], ws_ref[...])
              + b_ref[...])

    logits = _dot(xn, wo_ref[...]) + bo_ref[...]
    m = jnp.max(logits, axis=-1, keepdims=True)
    e = jnp.exp(logits - m)
    o_ref[...] = e / jnp.sum(e, axis=-1, keepdims=True)

    @pl.when((i == 0) & (sidx == 0))
    def _():
        loss_ref[...] = lprev_ref[...]

    @pl.when(sidx == 0)
    def _():
        dd = _dot(xn, wss_ref[...]) + bss_ref[...] - h_ref[...]
        loss_ref[...] = (loss_ref[...]
                         + (jnp.sum(dd * dd) * (SSW / (N * F)))[None, None])


def _tc_layer1_out(p, deg3, x, h, Wm, Ws, b2, Wss, bss2, W_out, bo2, lprev):
    nblk = N // _BT
    return pl.pallas_call(
        _tc_layer1_out_body,
        grid=(nblk, S),
        in_specs=[
            pl.BlockSpec((NC, _BT, H), lambda i, s: (0, i, 0)),
            pl.BlockSpec((NC, _BT, 1), lambda i, s: (0, i, 0)),
            pl.BlockSpec((_BT, H), lambda i, s: (i, 0)),
            pl.BlockSpec((_BT, F), lambda i, s: (i, 0)),
            pl.BlockSpec((H, H), lambda i, s: (0, 0)),
            pl.BlockSpec((H, H), lambda i, s: (0, 0)),
            pl.BlockSpec((1, H), lambda i, s: (0, 0)),
            pl.BlockSpec((H, F), lambda i, s: (0, 0)),
            pl.BlockSpec((1, F), lambda i, s: (0, 0)),
            pl.BlockSpec((H, C), lambda i, s: (0, 0)),
            pl.BlockSpec((1, C), lambda i, s: (0, 0)),
            pl.BlockSpec((1, 1), lambda i, s: (0, 0)),
        ],
        out_specs=[
            pl.BlockSpec((_BT, C), lambda i, s: (s * nblk + i, 0)),
            pl.BlockSpec((1, 1), lambda i, s: (0, 0)),
        ],
        out_shape=[
            jax.ShapeDtypeStruct((S * N, C), jnp.float32),
            jax.ShapeDtypeStruct((1, 1), jnp.float32),
        ],
    )(p, deg3, x, h, Wm, Ws, b2, Wss, bss2, W_out, bo2, lprev)


def kernel(h, edge_index, W_in, b_in, l0_Wm, l0_Ws, l0_b, l0_Wss, l0_bss,
           l1_Wm, l1_Ws, l1_b, l1_Wss, l1_bss, W_out, b_out):
    ei = edge_index.reshape(2 * E)
    pA, deg = _make_sc_pass(True)(h, ei)
    Wim, Wis, bim, bc = _tc_fuse(W_in, b_in.reshape(1, H), l0_Wm, l0_Ws,
                                 l0_b.reshape(1, H))
    deg3 = deg.reshape(NC, NP, 1)
    xn0, loss0 = _tc_layer0(pA, deg3, h, Wim, Wis, bim, bc,
                            l0_Wss, l0_bss.reshape(1, F))
    (pB,) = _make_sc_pass(False)(xn0, ei)
    sm, loss = _tc_layer1_out(pB, deg3, xn0, h, l1_Wm, l1_Ws,
                               l1_b.reshape(1, H), l1_Wss,
                               l1_bss.reshape(1, F), W_out,
                               b_out.reshape(1, C), loss0)
    return sm.reshape(S, N, C), loss[0, 0]


# speedup vs baseline: 1.0398x; 1.0398x over previous
import functools

import jax
import jax.numpy as jnp
from jax import lax
from jax.experimental import pallas as pl
from jax.experimental.pallas import tpu as pltpu
from jax.experimental.pallas import tpu_sc as plsc

N = 10000
NP = 10240
E = 320000
F = 128
H = 128
C = 64
S = 4
SSW = 0.05

NC = 2
NS = 16
NW = NC * NS
EW = E // NW
K = 80
NCH = EW // K
NB = 4
NI = 6
RPT = NP // NS

_HIGH = lax.Precision.DEFAULT


def _elu(v):
    return jnp.where(v > 0, v, jnp.exp(v) - 1.0)


@functools.lru_cache(maxsize=None)
def _make_sc_pass(with_deg):
    mesh = plsc.VectorSubcoreMesh(core_axis_name="c", subcore_axis_name="s",
                                  num_cores=NC, num_subcores=NS)
    out_type = [jax.ShapeDtypeStruct((NC, NP, H), jnp.float32)]
    if with_deg:
        out_type.append(jax.ShapeDtypeStruct((NC, NP), jnp.float32))

    scratch = [
        pltpu.VMEM((NI, 1, K), jnp.int32),
        pltpu.VMEM((NI, 1, K), jnp.int32),
        pltpu.VMEM((NB, K, H), jnp.float32),
        pltpu.VMEM((K,), jnp.float32),
        pltpu.VMEM_SHARED((NP, H), jnp.float32),
        pltpu.VMEM_SHARED((NP,), jnp.float32),
        pltpu.SemaphoreType.DMA,
        pltpu.SemaphoreType.DMA,
        pltpu.SemaphoreType.DMA,
        pltpu.SemaphoreType.DMA,
    ]

    def body(x_hbm, ei_hbm, *rest):
        if with_deg:
            part_hbm, deg_hbm = rest[0], rest[1]
            rest = rest[2:]
        else:
            part_hbm = rest[0]
            deg_hbm = None
            rest = rest[1:]
        sidx, didx, rows, ones, agg_sh, deg_sh, sem_i, sem_g, sem_s, sem_d = rest

        c = lax.axis_index("c")
        s = lax.axis_index("s")
        wid = c * NS + s
        r0 = s * RPT

        ebase = wid * EW

        def stage_idx(j, jx):
            e0 = ebase + j * K
            pltpu.async_copy(ei_hbm.at[pl.ds(e0, K)], sidx.at[jx, 0], sem_i)
            pltpu.async_copy(ei_hbm.at[pl.ds(E + e0, K)], didx.at[jx, 0], sem_i)

        def wait_idx(jx):
            pltpu.make_async_copy(ei_hbm.at[pl.ds(0, K)], sidx.at[jx, 0],
                                  sem_i).wait()
            pltpu.make_async_copy(ei_hbm.at[pl.ds(0, K)], didx.at[jx, 0],
                                  sem_i).wait()

        def issue_gather(jx, q):
            pltpu.async_copy(x_hbm.at[sidx.at[jx, 0]], rows.at[q], sem_g)

        def wait_gather_issue_scatter(jx, q):
            pltpu.make_async_copy(x_hbm.at[sidx.at[jx, 0]], rows.at[q],
                                  sem_g).wait()
            pltpu.async_copy(rows.at[q], agg_sh.at[didx.at[jx, 0]],
                             sem_s, add=True)
            if with_deg:
                pltpu.async_copy(ones, deg_sh.at[didx.at[jx, 0]],
                                 sem_d, add=True)

        def drain_scatter(jx, q):
            pltpu.make_async_copy(rows.at[q], agg_sh.at[didx.at[jx, 0]],
                                  sem_s).wait()
            if with_deg:
                pltpu.make_async_copy(ones, deg_sh.at[didx.at[jx, 0]],
                                      sem_d).wait()

        def chunk_step(js, jr=None, drain=True, stage=True, advance=True):
            if jr is None:
                jr = js
            q = js % NB
            jx = js % NI
            if drain:
                drain_scatter((js - 2) % NI, (js - 2) % NB)
            if stage:
                stage_idx(jr + 3, (js + 3) % NI)
            if advance:
                wait_idx((js + 2) % NI)
                issue_gather((js + 2) % NI, (js + 2) % NB)
            wait_gather_issue_scatter(jx, q)

        stage_idx(0, 0)
        stage_idx(1, 1)
        stage_idx(2, 2)

        @pl.loop(0, K)
        def _(i):
            @pl.loop(0, H, step=16)
            def _(j):
                rows.at[2, i, pl.ds(j, 16)][...] = jnp.zeros((16,), jnp.float32)

        @pl.loop(0, K, step=16)
        def _(j):
            ones.at[pl.ds(j, 16)][...] = jnp.ones((16,), jnp.float32)

        @pl.loop(0, RPT, step=K)
        def _(r):
            pltpu.sync_copy(rows.at[2], agg_sh.at[pl.ds(r0 + r, K)])

        if with_deg:
            @pl.loop(0, RPT, step=128)
            def _(r):
                pltpu.sync_copy(rows.at[2, 0], deg_sh.at[pl.ds(r0 + r, 128)])

        wait_idx(0)
        issue_gather(0, 0)
        wait_idx(1)
        issue_gather(1, 1)
        plsc.subcore_barrier()

        chunk_step(0, drain=False)
        chunk_step(1, drain=False)

        @pl.loop(2, NCH - 3, step=12)
        def _(jb):
            for u in range(12):
                chunk_step(2 + u, jb + u)

        chunk_step(NCH - 3, stage=False)
        chunk_step(NCH - 2, stage=False, advance=False)
        chunk_step(NCH - 1, stage=False, advance=False)
        drain_scatter((NCH - 2) % NI, (NCH - 2) % NB)
        drain_scatter((NCH - 1) % NI, (NCH - 1) % NB)

        plsc.subcore_barrier()

        pltpu.sync_copy(agg_sh.at[pl.ds(r0, RPT)], part_hbm.at[c, pl.ds(r0, RPT)])
        if with_deg:
            pltpu.sync_copy(deg_sh.at[pl.ds(r0, RPT)], deg_hbm.at[c, pl.ds(r0, RPT)])

    return pl.kernel(body, out_type=out_type, mesh=mesh, scratch_types=scratch)


_BT = 5000

_W2 = [pl.BlockSpec((H, H), lambda i: (0, 0))]
_P_SPEC = pl.BlockSpec((NC, _BT, H), lambda i: (0, i, 0))
_D_SPEC = pl.BlockSpec((NC, _BT, 1), lambda i: (0, i, 0))
_ROW_SPEC = pl.BlockSpec((_BT, H), lambda i: (i, 0))
_L_SPEC = pl.BlockSpec((1, 1), lambda i: (0, 0))


def _dot(a, b):
    return jnp.dot(a, b, precision=_HIGH, preferred_element_type=jnp.float32)


def _tc_fuse_body(win_ref, bin_ref, wm_ref, ws_ref, b_ref,
                  wim_ref, wis_ref, bim_ref, bc_ref):
    hi = lax.Precision.HIGHEST
    win = win_ref[...]
    bin_ = bin_ref[...]
    wim_ref[...] = jnp.dot(win, wm_ref[...], precision=hi,
                           preferred_element_type=jnp.float32)
    wis_ref[...] = jnp.dot(win, ws_ref[...], precision=hi,
                           preferred_element_type=jnp.float32)
    bim_ref[...] = jnp.dot(bin_, wm_ref[...], precision=hi,
                           preferred_element_type=jnp.float32)
    bc_ref[...] = (
        jnp.dot(bin_, ws_ref[...], precision=hi,
                preferred_element_type=jnp.float32)
        + b_ref[...]
    )


def _tc_fuse(W_in, b_in2, Wm, Ws, b2):
    w_spec = pl.BlockSpec((H, H), lambda: (0, 0))
    b_spec = pl.BlockSpec((1, H), lambda: (0, 0))
    return pl.pallas_call(
        _tc_fuse_body,
        in_specs=[w_spec, b_spec, w_spec, w_spec, b_spec],
        out_specs=[w_spec, w_spec, b_spec, b_spec],
        out_shape=[
            jax.ShapeDtypeStruct((H, H), jnp.float32),
            jax.ShapeDtypeStruct((H, H), jnp.float32),
            jax.ShapeDtypeStruct((1, H), jnp.float32),
            jax.ShapeDtypeStruct((1, H), jnp.float32),
        ],
    )(W_in, b_in2, Wm, Ws, b2)


def _tc_layer0_body(p_ref, d_ref, h_ref, wim_ref, wis_ref, bim_ref, bc_ref,
                    wss_ref, bss_ref, xn_ref, loss_ref):
    i = pl.program_id(0)
    dsum = d_ref[0] + d_ref[1]
    inv = 1.0 / jnp.maximum(dsum, 1.0)
    mask = jnp.minimum(dsum, 1.0)
    hbar = (p_ref[0] + p_ref[1]) * inv
    xn = _elu(_dot(hbar, wim_ref[...]) + _dot(h_ref[...], wis_ref[...])
              + bim_ref[...] * mask + bc_ref[...])
    xn_ref[...] = xn
    dd = _dot(xn, wss_ref[...]) + bss_ref[...] - h_ref[...]

    @pl.when(i == 0)
    def _():
        loss_ref[...] = jnp.zeros((1, 1), jnp.float32)

    loss_ref[...] = loss_ref[...] + (jnp.sum(dd * dd) * (SSW / (N * F)))[None, None]


def _tc_layer0(p, deg3, h, Wim, Wis, bim, bc, Wss, bss2):
    return pl.pallas_call(
        _tc_layer0_body,
        grid=(N // _BT,),
        in_specs=[
            _P_SPEC,
            _D_SPEC,
            _ROW_SPEC,
            pl.BlockSpec((H, H), lambda i: (0, 0)),
            pl.BlockSpec((H, H), lambda i: (0, 0)),
            pl.BlockSpec((1, H), lambda i: (0, 0)),
            pl.BlockSpec((1, H), lambda i: (0, 0)),
            pl.BlockSpec((H, F), lambda i: (0, 0)),
            pl.BlockSpec((1, F), lambda i: (0, 0)),
        ],
        out_specs=[_ROW_SPEC, _L_SPEC],
        out_shape=[
            jax.ShapeDtypeStruct((N, H), jnp.float32),
            jax.ShapeDtypeStruct((1, 1), jnp.float32),
        ],
    )(p, deg3, h, Wim, Wis, bim, bc, Wss, bss2)


def _tc_layer1_out_body(p_ref, d_ref, x_ref, h_ref, wm_ref, ws_ref, b_ref,
                        wss_ref, bss_ref, wo_ref, bo_ref, lprev_ref,
                        o_ref, loss_ref, sm_ref):
    i = pl.program_id(0)
    sidx = pl.program_id(1)

    @pl.when((i == 0) & (sidx == 0))
    def _():
        loss_ref[...] = lprev_ref[...]

    @pl.when(sidx == 0)
    def _():
        dsum = d_ref[0] + d_ref[1]
        inv = 1.0 / jnp.maximum(dsum, 1.0)
        agg = (p_ref[0] + p_ref[1]) * inv
        xn = _elu(_dot(agg, wm_ref[...]) + _dot(x_ref[...], ws_ref[...])
                  + b_ref[...])
        logits = _dot(xn, wo_ref[...]) + bo_ref[...]
        m = jnp.max(logits, axis=-1, keepdims=True)
        e = jnp.exp(logits - m)
        sm_ref[...] = e / jnp.sum(e, axis=-1, keepdims=True)
        dd = _dot(xn, wss_ref[...]) + bss_ref[...] - h_ref[...]
        loss_ref[...] = (loss_ref[...]
                         + (jnp.sum(dd * dd) * (SSW / (N * F)))[None, None])

    o_ref[...] = sm_ref[...]


def _tc_layer1_out(p, deg3, x, h, Wm, Ws, b2, Wss, bss2, W_out, bo2, lprev):
    nblk = N // _BT
    return pl.pallas_call(
        _tc_layer1_out_body,
        grid=(nblk, S),
        in_specs=[
            pl.BlockSpec((NC, _BT, H), lambda i, s: (0, i, 0)),
            pl.BlockSpec((NC, _BT, 1), lambda i, s: (0, i, 0)),
            pl.BlockSpec((_BT, H), lambda i, s: (i, 0)),
            pl.BlockSpec((_BT, F), lambda i, s: (i, 0)),
            pl.BlockSpec((H, H), lambda i, s: (0, 0)),
            pl.BlockSpec((H, H), lambda i, s: (0, 0)),
            pl.BlockSpec((1, H), lambda i, s: (0, 0)),
            pl.BlockSpec((H, F), lambda i, s: (0, 0)),
            pl.BlockSpec((1, F), lambda i, s: (0, 0)),
            pl.BlockSpec((H, C), lambda i, s: (0, 0)),
            pl.BlockSpec((1, C), lambda i, s: (0, 0)),
            pl.BlockSpec((1, 1), lambda i, s: (0, 0)),
        ],
        out_specs=[
            pl.BlockSpec((_BT, C), lambda i, s: (s * nblk + i, 0)),
            pl.BlockSpec((1, 1), lambda i, s: (0, 0)),
        ],
        out_shape=[
            jax.ShapeDtypeStruct((S * N, C), jnp.float32),
            jax.ShapeDtypeStruct((1, 1), jnp.float32),
        ],
        scratch_shapes=[pltpu.VMEM((_BT, C), jnp.float32)],
    )(p, deg3, x, h, Wm, Ws, b2, Wss, bss2, W_out, bo2, lprev)


def kernel(h, edge_index, W_in, b_in, l0_Wm, l0_Ws, l0_b, l0_Wss, l0_bss,
           l1_Wm, l1_Ws, l1_b, l1_Wss, l1_bss, W_out, b_out):
    ei = edge_index.reshape(2 * E)
    pA, deg = _make_sc_pass(True)(h, ei)
    Wim, Wis, bim, bc = _tc_fuse(W_in, b_in.reshape(1, H), l0_Wm, l0_Ws,
                                 l0_b.reshape(1, H))
    deg3 = deg.reshape(NC, NP, 1)
    xn0, loss0 = _tc_layer0(pA, deg3, h, Wim, Wis, bim, bc,
                            l0_Wss, l0_bss.reshape(1, F))
    (pB,) = _make_sc_pass(False)(xn0, ei)
    sm, loss = _tc_layer1_out(pB, deg3, xn0, h, l1_Wm, l1_Ws,
                               l1_b.reshape(1, H), l1_Wss,
                               l1_bss.reshape(1, F), W_out,
                               b_out.reshape(1, C), loss0)
    return sm.reshape(S, N, C), loss[0, 0]


# speedup vs baseline: 1.0973x; 1.0553x over previous
import functools

import jax
import jax.numpy as jnp
from jax import lax
from jax.experimental import pallas as pl
from jax.experimental.pallas import tpu as pltpu
from jax.experimental.pallas import tpu_sc as plsc

N = 10000
NP = 10240
E = 320000
F = 128
H = 128
C = 64
S = 4
SSW = 0.05

NC = 2
NS = 16
NW = NC * NS
EW = E // NW
K = 80
NCH = EW // K
NB = 4
NI = 6
RPT = NP // NS

_HIGH = lax.Precision.DEFAULT


def _elu(v):
    return jnp.where(v > 0, v, jnp.exp(v) - 1.0)


@functools.lru_cache(maxsize=None)
def _make_sc_pass(with_deg):
    mesh = plsc.VectorSubcoreMesh(core_axis_name="c", subcore_axis_name="s",
                                  num_cores=NC, num_subcores=NS)
    out_type = [jax.ShapeDtypeStruct((NC, NP, H), jnp.float32)]
    if with_deg:
        out_type.append(jax.ShapeDtypeStruct((NC, NP), jnp.float32))

    scratch = [
        pltpu.VMEM((NI, 1, K), jnp.int32),
        pltpu.VMEM((NI, 1, K), jnp.int32),
        pltpu.VMEM((NB, K, H), jnp.float32),
        pltpu.VMEM((K,), jnp.float32),
        pltpu.VMEM_SHARED((NP, H), jnp.float32),
        pltpu.VMEM_SHARED((NP,), jnp.float32),
        pltpu.SemaphoreType.DMA,
        pltpu.SemaphoreType.DMA,
        pltpu.SemaphoreType.DMA,
        pltpu.SemaphoreType.DMA,
    ]

    def body(x_hbm, ei_hbm, *rest):
        if with_deg:
            part_hbm, deg_hbm = rest[0], rest[1]
            rest = rest[2:]
        else:
            part_hbm = rest[0]
            deg_hbm = None
            rest = rest[1:]
        sidx, didx, rows, ones, agg_sh, deg_sh, sem_i, sem_g, sem_s, sem_d = rest

        c = lax.axis_index("c")
        s = lax.axis_index("s")
        wid = c * NS + s
        r0 = s * RPT

        ebase = wid * EW

        def stage_idx(j, jx):
            e0 = ebase + j * K
            pltpu.async_copy(ei_hbm.at[pl.ds(e0, K)], sidx.at[jx, 0], sem_i)
            pltpu.async_copy(ei_hbm.at[pl.ds(E + e0, K)], didx.at[jx, 0], sem_i)

        def wait_idx(jx):
            pltpu.make_async_copy(ei_hbm.at[pl.ds(0, K)], sidx.at[jx, 0],
                                  sem_i).wait()
            pltpu.make_async_copy(ei_hbm.at[pl.ds(0, K)], didx.at[jx, 0],
                                  sem_i).wait()

        def issue_gather(jx, q):
            pltpu.async_copy(x_hbm.at[sidx.at[jx, 0]], rows.at[q], sem_g)

        def wait_gather_issue_scatter(jx, q):
            pltpu.make_async_copy(x_hbm.at[sidx.at[jx, 0]], rows.at[q],
                                  sem_g).wait()
            pltpu.async_copy(rows.at[q], agg_sh.at[didx.at[jx, 0]],
                             sem_s, add=True)
            if with_deg:
                pltpu.async_copy(ones, deg_sh.at[didx.at[jx, 0]],
                                 sem_d, add=True)

        def drain_scatter(jx, q):
            pltpu.make_async_copy(rows.at[q], agg_sh.at[didx.at[jx, 0]],
                                  sem_s).wait()
            if with_deg:
                pltpu.make_async_copy(ones, deg_sh.at[didx.at[jx, 0]],
                                      sem_d).wait()

        def chunk_step(js, jr=None, drain=True, stage=True, advance=True):
            if jr is None:
                jr = js
            q = js % NB
            jx = js % NI
            if drain:
                drain_scatter((js - 2) % NI, (js - 2) % NB)
            if stage:
                stage_idx(jr + 3, (js + 3) % NI)
            if advance:
                wait_idx((js + 2) % NI)
                issue_gather((js + 2) % NI, (js + 2) % NB)
            wait_gather_issue_scatter(jx, q)

        stage_idx(0, 0)
        stage_idx(1, 1)
        stage_idx(2, 2)

        @pl.loop(0, K)
        def _(i):
            @pl.loop(0, H, step=16)
            def _(j):
                rows.at[2, i, pl.ds(j, 16)][...] = jnp.zeros((16,), jnp.float32)

        @pl.loop(0, K, step=16)
        def _(j):
            ones.at[pl.ds(j, 16)][...] = jnp.ones((16,), jnp.float32)

        @pl.loop(0, RPT, step=K)
        def _(r):
            pltpu.sync_copy(rows.at[2], agg_sh.at[pl.ds(r0 + r, K)])

        if with_deg:
            @pl.loop(0, RPT, step=128)
            def _(r):
                pltpu.sync_copy(rows.at[2, 0], deg_sh.at[pl.ds(r0 + r, 128)])

        wait_idx(0)
        issue_gather(0, 0)
        wait_idx(1)
        issue_gather(1, 1)
        plsc.subcore_barrier()

        chunk_step(0, drain=False)
        chunk_step(1, drain=False)

        @pl.loop(2, NCH - 3, step=12)
        def _(jb):
            for u in range(12):
                chunk_step(2 + u, jb + u)

        chunk_step(NCH - 3, stage=False)
        chunk_step(NCH - 2, stage=False, advance=False)
        chunk_step(NCH - 1, stage=False, advance=False)
        drain_scatter((NCH - 2) % NI, (NCH - 2) % NB)
        drain_scatter((NCH - 1) % NI, (NCH - 1) % NB)

        plsc.subcore_barrier()

        pltpu.sync_copy(agg_sh.at[pl.ds(r0, RPT)], part_hbm.at[c, pl.ds(r0, RPT)])
        if with_deg:
            pltpu.sync_copy(deg_sh.at[pl.ds(r0, RPT)], deg_hbm.at[c, pl.ds(r0, RPT)])

    return pl.kernel(body, out_type=out_type, mesh=mesh, scratch_types=scratch)


_BT = 5000

_W2 = [pl.BlockSpec((H, H), lambda i: (0, 0))]
_P_SPEC = pl.BlockSpec((NC, _BT, H), lambda i: (0, i, 0))
_D_SPEC = pl.BlockSpec((NC, _BT, 1), lambda i: (0, i, 0))
_ROW_SPEC = pl.BlockSpec((_BT, H), lambda i: (i, 0))
_L_SPEC = pl.BlockSpec((1, 1), lambda i: (0, 0))


def _dot(a, b):
    return jnp.dot(a, b, precision=_HIGH, preferred_element_type=jnp.float32)


def _tc_fuse_body(win_ref, bin_ref, wm_ref, ws_ref, b_ref,
                  wim_ref, wis_ref, bim_ref, bc_ref):
    hi = lax.Precision.HIGHEST
    win = win_ref[...]
    bin_ = bin_ref[...]
    wim_ref[...] = jnp.dot(win, wm_ref[...], precision=hi,
                           preferred_element_type=jnp.float32)
    wis_ref[...] = jnp.dot(win, ws_ref[...], precision=hi,
                           preferred_element_type=jnp.float32)
    bim_ref[...] = jnp.dot(bin_, wm_ref[...], precision=hi,
                           preferred_element_type=jnp.float32)
    bc_ref[...] = (
        jnp.dot(bin_, ws_ref[...], precision=hi,
                preferred_element_type=jnp.float32)
        + b_ref[...]
    )


def _tc_fuse(W_in, b_in2, Wm, Ws, b2):
    w_spec = pl.BlockSpec((H, H), lambda: (0, 0))
    b_spec = pl.BlockSpec((1, H), lambda: (0, 0))
    return pl.pallas_call(
        _tc_fuse_body,
        in_specs=[w_spec, b_spec, w_spec, w_spec, b_spec],
        out_specs=[w_spec, w_spec, b_spec, b_spec],
        out_shape=[
            jax.ShapeDtypeStruct((H, H), jnp.float32),
            jax.ShapeDtypeStruct((H, H), jnp.float32),
            jax.ShapeDtypeStruct((1, H), jnp.float32),
            jax.ShapeDtypeStruct((1, H), jnp.float32),
        ],
    )(W_in, b_in2, Wm, Ws, b2)


def _tc_layer0_body(p_ref, d_ref, h_ref, wim_ref, wis_ref, bim_ref, bc_ref,
                    wss_ref, bss_ref, xn_ref, loss_ref):
    i = pl.program_id(0)
    dsum = d_ref[0] + d_ref[1]
    inv = 1.0 / jnp.maximum(dsum, 1.0)
    mask = jnp.minimum(dsum, 1.0)
    hbar = (p_ref[0] + p_ref[1]) * inv
    xn = _elu(_dot(hbar, wim_ref[...]) + _dot(h_ref[...], wis_ref[...])
              + bim_ref[...] * mask + bc_ref[...])
    xn_ref[...] = xn
    dd = _dot(xn, wss_ref[...]) + bss_ref[...] - h_ref[...]

    @pl.when(i == 0)
    def _():
        loss_ref[...] = jnp.zeros((1, 1), jnp.float32)

    loss_ref[...] = loss_ref[...] + (jnp.sum(dd * dd) * (SSW / (N * F)))[None, None]


def _tc_layer0(p, deg3, h, Wim, Wis, bim, bc, Wss, bss2):
    return pl.pallas_call(
        _tc_layer0_body,
        grid=(N // _BT,),
        in_specs=[
            _P_SPEC,
            _D_SPEC,
            _ROW_SPEC,
            pl.BlockSpec((H, H), lambda i: (0, 0)),
            pl.BlockSpec((H, H), lambda i: (0, 0)),
            pl.BlockSpec((1, H), lambda i: (0, 0)),
            pl.BlockSpec((1, H), lambda i: (0, 0)),
            pl.BlockSpec((H, F), lambda i: (0, 0)),
            pl.BlockSpec((1, F), lambda i: (0, 0)),
        ],
        out_specs=[_ROW_SPEC, _L_SPEC],
        out_shape=[
            jax.ShapeDtypeStruct((N, H), jnp.float32),
            jax.ShapeDtypeStruct((1, 1), jnp.float32),
        ],
    )(p, deg3, h, Wim, Wis, bim, bc, Wss, bss2)


def _tc_layer1_out_body(p_ref, d_ref, x_ref, h_ref, wm_ref, ws_ref, b_ref,
                        wss_ref, bss_ref, wo_ref, bo_ref, lprev_ref,
                        o_ref, loss_ref):
    i = pl.program_id(0)
    dsum = d_ref[0] + d_ref[1]
    inv = 1.0 / jnp.maximum(dsum, 1.0)
    agg = (p_ref[0] + p_ref[1]) * inv
    xn = _elu(_dot(agg, wm_ref[...]) + _dot(x_ref[...], ws_ref[...])
              + b_ref[...])
    dd = _dot(xn, wss_ref[...]) + bss_ref[...] - h_ref[...]

    logits = _dot(xn, wo_ref[...]) + bo_ref[...]
    m = jnp.max(logits, axis=-1, keepdims=True)
    e = jnp.exp(logits - m)
    o_ref[...] = e / jnp.sum(e, axis=-1, keepdims=True)

    @pl.when(i == 0)
    def _():
        loss_ref[...] = lprev_ref[...]

    loss_ref[...] = loss_ref[...] + (jnp.sum(dd * dd) * (SSW / (N * F)))[None, None]


def _tc_layer1_out(p, deg3, x, h, Wm, Ws, b2, Wss, bss2, W_out, bo2, lprev):
    return pl.pallas_call(
        _tc_layer1_out_body,
        grid=(N // _BT,),
        in_specs=[
            _P_SPEC,
            _D_SPEC,
            _ROW_SPEC,
            _ROW_SPEC,
            pl.BlockSpec((H, H), lambda i: (0, 0)),
            pl.BlockSpec((H, H), lambda i: (0, 0)),
            pl.BlockSpec((1, H), lambda i: (0, 0)),
            pl.BlockSpec((H, F), lambda i: (0, 0)),
            pl.BlockSpec((1, F), lambda i: (0, 0)),
            pl.BlockSpec((H, C), lambda i: (0, 0)),
            pl.BlockSpec((1, C), lambda i: (0, 0)),
            _L_SPEC,
        ],
        out_specs=[
            pl.BlockSpec((_BT, C), lambda i: (i, 0)),
            _L_SPEC,
        ],
        out_shape=[
            jax.ShapeDtypeStruct((N, C), jnp.float32),
            jax.ShapeDtypeStruct((1, 1), jnp.float32),
        ],
    )(p, deg3, x, h, Wm, Ws, b2, Wss, bss2, W_out, bo2, lprev)


def kernel(h, edge_index, W_in, b_in, l0_Wm, l0_Ws, l0_b, l0_Wss, l0_bss,
           l1_Wm, l1_Ws, l1_b, l1_Wss, l1_bss, W_out, b_out):
    ei = edge_index.reshape(2 * E)
    pA, deg = _make_sc_pass(True)(h, ei)
    Wim, Wis, bim, bc = _tc_fuse(W_in, b_in.reshape(1, H), l0_Wm, l0_Ws,
                                 l0_b.reshape(1, H))
    deg3 = deg.reshape(NC, NP, 1)
    xn0, loss0 = _tc_layer0(pA, deg3, h, Wim, Wis, bim, bc,
                            l0_Wss, l0_bss.reshape(1, F))
    (pB,) = _make_sc_pass(False)(xn0, ei)
    sm, loss = _tc_layer1_out(pB, deg3, xn0, h, l1_Wm, l1_Ws,
                               l1_b.reshape(1, H), l1_Wss,
                               l1_bss.reshape(1, F), W_out,
                               b_out.reshape(1, C), loss0)
    return jnp.broadcast_to(sm[None], (S, N, C)), loss[0, 0]
